# baseline (speedup 1.0000x reference)
import functools

import jax
import jax.numpy as jnp
from jax import lax
from jax.experimental import pallas as pl
from jax.experimental.pallas import tpu as pltpu
from jax.experimental.pallas import tpu_sc as plsc

HIDDEN = 64
SEQ = 2048
BATCH = 4
TOTAL = BATCH * SEQ
NC, NS = 2, 16
NW = NC * NS
B_PER_W = TOTAL // NW
TILE = 8


def _make_kernel():
    mesh = plsc.VectorSubcoreMesh(core_axis_name="c", subcore_axis_name="s")

    out_rows = B_PER_W * HIDDEN // 128

    @functools.partial(
        pl.kernel,
        mesh=mesh,
        compiler_params=pltpu.CompilerParams(needs_layout_passes=False),
        out_type=jax.ShapeDtypeStruct((TOTAL * HIDDEN // 128, 128), jnp.float32),
        scratch_types=[
            pltpu.VMEM((TOTAL // 128, 128), jnp.int32),
            pltpu.VMEM((2, 32, TILE, HIDDEN), jnp.float32),
            pltpu.VMEM((out_rows, 128), jnp.float32),
            pltpu.SemaphoreType.DMA,
            pltpu.SemaphoreType.DMA,
        ],
    )
    def body(x_hbm, emb_hbm, pos_hbm, out_hbm, idx_v, tiles_v, pos_v,
             sem_a, sem_b):
        wid = lax.axis_index("s") * NC + lax.axis_index("c")

        pltpu.sync_copy(x_hbm, idx_v)
        pos_base = pl.multiple_of(
            lax.rem(wid, SEQ // B_PER_W) * out_rows, out_rows)
        pltpu.sync_copy(pos_hbm.at[pl.ds(pos_base, out_rows)], pos_v)

        scale = jnp.float32(8.0)

        sems = (sem_a, sem_b)
        n_groups = B_PER_W // 32

        def fire(g):
            row = wid * 2 + (g >> 2)
            col = (g & 3) * 32
            xs = []
            for h in range(2):
                xg = idx_v[row, pl.ds(col + h * 16, 16)]
                xs += [xg[l] for l in range(16)]
            slot = g & 1
            return xs, [
                pltpu.async_copy(
                    emb_hbm.at[xs[l] >> 3], tiles_v.at[slot, l], sems[slot])
                for l in range(32)
            ]

        pending = fire(0)
        for g in range(n_groups):
            slot = g & 1
            xs, copies = pending
            if g + 1 < n_groups:
                pending = fire(g + 1)
            for l in range(32):
                copies[l].wait()
                r7 = xs[l] & 7
                r = g * 16 + (l >> 1)
                for j in range(HIDDEN // 16):
                    sl = pl.ds((l & 1) * HIDDEN + j * 16, 16)
                    gv = tiles_v[slot, l, r7, pl.ds(j * 16, 16)]
                    pos_v[r, sl] = gv * scale + pos_v[r, sl]

        out_base = pl.multiple_of(wid * out_rows, out_rows)
        pltpu.sync_copy(pos_v, out_hbm.at[pl.ds(out_base, out_rows)])

    return body


def kernel(x, emb_table, pos_table):
    xf = x.reshape(TOTAL // 128, 128).astype(jnp.int32)
    emb3 = emb_table.reshape(emb_table.shape[0] // TILE, TILE, HIDDEN)
    pos2 = pos_table.reshape(SEQ * HIDDEN // 128, 128)
    out = _make_kernel()(xf, emb3, pos2)
    return out.reshape(BATCH, SEQ, HIDDEN)


# speedup vs baseline: 1.0323x; 1.0323x over previous
import functools

import jax
import jax.numpy as jnp
from jax import lax
from jax.experimental import pallas as pl
from jax.experimental.pallas import tpu as pltpu
from jax.experimental.pallas import tpu_sc as plsc

HIDDEN = 64
SEQ = 2048
BATCH = 4
TOTAL = BATCH * SEQ
NC, NS = 2, 16
NW = NC * NS
B_PER_W = TOTAL // NW
TILE = 8


def _make_kernel():
    mesh = plsc.VectorSubcoreMesh(core_axis_name="c", subcore_axis_name="s")

    out_rows = B_PER_W * HIDDEN // 128

    @functools.partial(
        pl.kernel,
        mesh=mesh,
        compiler_params=pltpu.CompilerParams(needs_layout_passes=False),
        out_type=jax.ShapeDtypeStruct((TOTAL * HIDDEN // 128, 128), jnp.float32),
        scratch_types=[
            pltpu.VMEM((TOTAL // 128, 128), jnp.int32),
            pltpu.VMEM((32, TILE, HIDDEN), jnp.float32),
            pltpu.VMEM((out_rows, 128), jnp.float32),
            pltpu.SemaphoreType.DMA,
        ],
    )
    def body(x_hbm, emb_hbm, pos_hbm, out_hbm, idx_v, tiles_v, pos_v, sem):
        wid = lax.axis_index("s") * NC + lax.axis_index("c")

        pltpu.sync_copy(x_hbm, idx_v)
        pos_base = pl.multiple_of(
            lax.rem(wid, SEQ // B_PER_W) * out_rows, out_rows)
        pltpu.sync_copy(pos_hbm.at[pl.ds(pos_base, out_rows)], pos_v)

        scale = jnp.float32(8.0)

        def step(gi, carry):
            row = wid * 2 + (gi >> 2)
            col = (gi & 3) * 32
            xs = []
            for h in range(2):
                xg = idx_v[row, pl.ds(col + h * 16, 16)]
                xs += [xg[l] for l in range(16)]
            copies = [
                pltpu.async_copy(emb_hbm.at[xs[l] >> 3], tiles_v.at[l], sem)
                for l in range(32)
            ]
            for l in range(32):
                copies[l].wait()
                r7 = xs[l] & 7
                r = gi * 16 + (l >> 1)
                for j in range(HIDDEN // 16):
                    sl = pl.ds((l & 1) * HIDDEN + j * 16, 16)
                    g = tiles_v[l, r7, pl.ds(j * 16, 16)]
                    pos_v[r, sl] = g * scale + pos_v[r, sl]
            return carry

        lax.fori_loop(0, B_PER_W // 32, step, 0)

        out_base = pl.multiple_of(wid * out_rows, out_rows)
        pltpu.sync_copy(pos_v, out_hbm.at[pl.ds(out_base, out_rows)])

    return body


def kernel(x, emb_table, pos_table):
    xf = x.reshape(TOTAL // 128, 128).astype(jnp.int32)
    emb3 = emb_table.reshape(emb_table.shape[0] // TILE, TILE, HIDDEN)
    pos2 = pos_table.reshape(SEQ * HIDDEN // 128, 128)
    out = _make_kernel()(xf, emb3, pos2)
    return out.reshape(BATCH, SEQ, HIDDEN)
